# baseline (speedup 1.0000x reference)
import jax
import jax.numpy as jnp
from jax import lax
from jax.experimental import pallas as pl
from jax.experimental.pallas import tpu as pltpu
from jax.experimental.pallas import tpu_sc as plsc

_S = 128
_B = 4096
_D = 256
_WEIGHT = 0.05
_NC = 2
_NS = 16
_RPC = _B // _NC
_L = 16
_G = _RPC // _L


def _sc_body(emb_hbm, sid_hbm, out_sums, out_qc, sid_v, stripe_v, acc_s, acc_qc):
    cid = lax.axis_index("c")
    tid = lax.axis_index("s")
    rbase = cid * _RPC
    cbase = tid * _L

    iota = lax.broadcasted_iota(jnp.int32, (_L,), 0)
    zv = jnp.zeros((_L,), jnp.float32)
    ones = jnp.full((_L,), 1.0, jnp.float32)

    pltpu.sync_copy(sid_hbm.at[pl.ds(rbase, _RPC)], sid_v)
    pltpu.sync_copy(emb_hbm.at[pl.ds(rbase, _RPC), pl.ds(cbase, _L)], stripe_v)

    def zero_body(r, carry):
        acc_s[r, pl.ds(0, _L)] = zv
        acc_qc[r, pl.ds(0, _L)] = zv
        acc_qc[r, pl.ds(_L, _L)] = zv
        return carry

    lax.fori_loop(0, _S, zero_body, 0)

    def group_body(g, carry):
        sv = sid_v[pl.ds(g * _L, _L)]
        rowv = g * _L + iota
        plsc.addupdate_scatter(acc_qc, [sv, iota + _L], ones)
        for d in range(_L):
            colv = jnp.bitwise_and(iota + d, _L - 1)
            v = plsc.load_gather(stripe_v, [rowv, colv])
            plsc.addupdate_scatter(acc_s, [sv, colv], v)
            plsc.addupdate_scatter(acc_qc, [sv, iota], v * v)
        return carry

    lax.fori_loop(0, _G, group_body, 0)

    pltpu.sync_copy(acc_s, out_sums.at[cid, :, pl.ds(cbase, _L)])
    pltpu.sync_copy(acc_qc, out_qc.at[cid, tid])


_sc_call = pl.kernel(
    _sc_body,
    out_type=(jax.ShapeDtypeStruct((_NC, _S, _D), jnp.float32),
              jax.ShapeDtypeStruct((_NC, _NS, _S, 2 * _L), jnp.float32)),
    mesh=plsc.VectorSubcoreMesh(core_axis_name="c", subcore_axis_name="s"),
    compiler_params=pltpu.CompilerParams(use_tc_tiling_on_sc=False,
                                         needs_layout_passes=False),
    scratch_types=[
        pltpu.VMEM((_RPC,), jnp.int32),
        pltpu.VMEM((_RPC, _L), jnp.float32),
        pltpu.VMEM((_S, _L), jnp.float32),
        pltpu.VMEM((_S, 2 * _L), jnp.float32),
    ],
)


def _combine_body(sums_ref, qc_ref, out_ref):
    sums = sums_ref[0] + sums_ref[1]
    qc = jnp.sum(qc_ref[:], axis=0)
    ssq = jnp.sum(qc[:, 0:_L], axis=1)
    counts = jnp.sum(qc[:, _L:], axis=1) * (1.0 / _NS)
    safe = jnp.maximum(counts, 1.0)
    var = (ssq - jnp.sum(sums * sums, axis=1) / safe) / (safe * _D)
    mask = (counts > 1.0).astype(jnp.float32)
    total = jnp.sum(var * mask)
    nsc = jnp.sum(mask)
    loss = jnp.where(nsc > 0, _WEIGHT * total / jnp.maximum(nsc, 1.0), 0.0)
    out_ref[:] = jnp.reshape(loss, (1, 1))


def kernel(embeddings, scaffolds, batch):
    del batch
    sums2, qc = _sc_call(embeddings, scaffolds.astype(jnp.int32))
    out = pl.pallas_call(
        _combine_body,
        out_shape=jax.ShapeDtypeStruct((1, 1), jnp.float32),
    )(sums2, qc.reshape(_NC * _NS, _S, 2 * _L))
    return out[0, 0]


# speedup vs baseline: 1.1339x; 1.1339x over previous
import jax
import jax.numpy as jnp
from jax import lax
from jax.experimental import pallas as pl
from jax.experimental.pallas import tpu as pltpu
from jax.experimental.pallas import tpu_sc as plsc

_S = 128
_B = 4096
_D = 256
_WEIGHT = 0.05
_NC = 2
_NS = 16
_RPC = _B // _NC
_L = 16
_G = _RPC // _L


def _sc_body(emb_hbm, sid_hbm, out_sums, out_qc, sid_v, stripe_v, acc_s, acc_qc):
    cid = lax.axis_index("c")
    tid = lax.axis_index("s")
    rbase = cid * _RPC
    cbase = tid * _L

    iota = lax.broadcasted_iota(jnp.int32, (_L,), 0)
    zv = jnp.zeros((_L,), jnp.float32)
    ones = jnp.full((_L,), 1.0, jnp.float32)

    pltpu.sync_copy(sid_hbm.at[pl.ds(rbase, _RPC)], sid_v)
    pltpu.sync_copy(emb_hbm.at[pl.ds(rbase, _RPC), pl.ds(cbase, _L)], stripe_v)

    @plsc.parallel_loop(0, _S, unroll=4)
    def _(r):
        acc_s[r, pl.ds(0, _L)] = zv
        acc_qc[r, pl.ds(0, _L)] = zv
        acc_qc[r, pl.ds(_L, _L)] = zv

    @plsc.parallel_loop(0, _G, unroll=2)
    def _(g):
        sv = sid_v[pl.ds(g * _L, _L)]
        rowv = g * _L + iota
        plsc.addupdate_scatter(acc_qc, [sv, iota + _L], ones)
        for d in range(_L):
            colv = jnp.bitwise_and(iota + d, _L - 1)
            v = plsc.load_gather(stripe_v, [rowv, colv])
            plsc.addupdate_scatter(acc_s, [sv, colv], v)
            plsc.addupdate_scatter(acc_qc, [sv, iota], v * v)

    pltpu.sync_copy(acc_s, out_sums.at[cid, :, pl.ds(cbase, _L)])
    pltpu.sync_copy(acc_qc, out_qc.at[cid, tid])


_sc_call = pl.kernel(
    _sc_body,
    out_type=(jax.ShapeDtypeStruct((_NC, _S, _D), jnp.float32),
              jax.ShapeDtypeStruct((_NC, _NS, _S, 2 * _L), jnp.float32)),
    mesh=plsc.VectorSubcoreMesh(core_axis_name="c", subcore_axis_name="s"),
    compiler_params=pltpu.CompilerParams(use_tc_tiling_on_sc=False,
                                         needs_layout_passes=False),
    scratch_types=[
        pltpu.VMEM((_RPC,), jnp.int32),
        pltpu.VMEM((_RPC, _L), jnp.float32),
        pltpu.VMEM((_S, _L), jnp.float32),
        pltpu.VMEM((_S, 2 * _L), jnp.float32),
    ],
)


def _combine_body(sums_ref, qc_ref, out_ref):
    sums = sums_ref[0] + sums_ref[1]
    qc = jnp.sum(qc_ref[:], axis=(0, 1))
    ssq = jnp.sum(qc[:, 0:_L], axis=1)
    counts = jnp.sum(qc[:, _L:], axis=1) * (1.0 / _NS)
    safe = jnp.maximum(counts, 1.0)
    var = (ssq - jnp.sum(sums * sums, axis=1) / safe) / (safe * _D)
    mask = (counts > 1.0).astype(jnp.float32)
    total = jnp.sum(var * mask)
    nsc = jnp.sum(mask)
    loss = jnp.where(nsc > 0, _WEIGHT * total / jnp.maximum(nsc, 1.0), 0.0)
    out_ref[:] = jnp.reshape(loss, (1, 1))


def kernel(embeddings, scaffolds, batch):
    del batch
    sums2, qc = _sc_call(embeddings, scaffolds.astype(jnp.int32))
    out = pl.pallas_call(
        _combine_body,
        out_shape=jax.ShapeDtypeStruct((1, 1), jnp.float32),
    )(sums2, qc)
    return out[0, 0]


# speedup vs baseline: 1.1773x; 1.0383x over previous
import jax
import jax.numpy as jnp
from jax import lax
from jax.experimental import pallas as pl
from jax.experimental.pallas import tpu as pltpu
from jax.experimental.pallas import tpu_sc as plsc

_S = 128
_B = 4096
_D = 256
_WEIGHT = 0.05
_NC = 2
_NS = 16
_RPC = _B // _NC
_L = 16
_G = _RPC // _L


def _sc_body(emb_hbm, sid_hbm, out_sums, out_qc, sid_v, stripe_v, acc_s, acc_qc):
    cid = lax.axis_index("c")
    tid = lax.axis_index("s")
    rbase = cid * _RPC
    cbase = tid * _L

    iota = lax.broadcasted_iota(jnp.int32, (_L,), 0)
    zv = jnp.zeros((_L,), jnp.float32)
    ones = jnp.full((_L,), 1.0, jnp.float32)

    pltpu.sync_copy(sid_hbm.at[pl.ds(rbase, _RPC)], sid_v)
    pltpu.sync_copy(emb_hbm.at[pl.ds(rbase, _RPC), pl.ds(cbase, _L)], stripe_v)

    @plsc.parallel_loop(0, _S, unroll=4)
    def _(r):
        acc_s[r, pl.ds(0, _L)] = zv
        acc_qc[r, pl.ds(0, _L)] = zv
        acc_qc[r, pl.ds(_L, _L)] = zv

    @plsc.parallel_loop(0, _G, unroll=4)
    def _(g):
        sv = sid_v[pl.ds(g * _L, _L)]
        rowv = g * _L + iota
        plsc.addupdate_scatter(acc_qc, [sv, iota + _L], ones)
        for d in range(_L):
            colv = jnp.bitwise_and(iota + d, _L - 1)
            v = plsc.load_gather(stripe_v, [rowv, colv])
            plsc.addupdate_scatter(acc_s, [sv, colv], v)
            plsc.addupdate_scatter(acc_qc, [sv, iota], v * v)

    pltpu.sync_copy(acc_s, out_sums.at[cid, :, pl.ds(cbase, _L)])
    pltpu.sync_copy(acc_qc, out_qc.at[cid, tid])


_sc_call = pl.kernel(
    _sc_body,
    out_type=(jax.ShapeDtypeStruct((_NC, _S, _D), jnp.float32),
              jax.ShapeDtypeStruct((_NC, _NS, _S, 2 * _L), jnp.float32)),
    mesh=plsc.VectorSubcoreMesh(core_axis_name="c", subcore_axis_name="s"),
    compiler_params=pltpu.CompilerParams(use_tc_tiling_on_sc=False,
                                         needs_layout_passes=False),
    scratch_types=[
        pltpu.VMEM((_RPC,), jnp.int32),
        pltpu.VMEM((_RPC, _L), jnp.float32),
        pltpu.VMEM((_S, _L), jnp.float32),
        pltpu.VMEM((_S, 2 * _L), jnp.float32),
    ],
)


def _combine_body(sums_ref, qc_ref, out_ref):
    sums = sums_ref[0] + sums_ref[1]
    qc = jnp.sum(qc_ref[:], axis=(0, 1))
    ssq = jnp.sum(qc[:, 0:_L], axis=1)
    counts = jnp.sum(qc[:, _L:], axis=1) * (1.0 / _NS)
    safe = jnp.maximum(counts, 1.0)
    var = (ssq - jnp.sum(sums * sums, axis=1) / safe) / (safe * _D)
    mask = (counts > 1.0).astype(jnp.float32)
    total = jnp.sum(var * mask)
    nsc = jnp.sum(mask)
    loss = jnp.where(nsc > 0, _WEIGHT * total / jnp.maximum(nsc, 1.0), 0.0)
    out_ref[:] = jnp.reshape(loss, (1, 1))


def kernel(embeddings, scaffolds, batch):
    del batch
    sums2, qc = _sc_call(embeddings, scaffolds.astype(jnp.int32))
    out = pl.pallas_call(
        _combine_body,
        out_shape=jax.ShapeDtypeStruct((1, 1), jnp.float32),
    )(sums2, qc)
    return out[0, 0]


# speedup vs baseline: 1.3397x; 1.1380x over previous
import jax
import jax.numpy as jnp
from jax import lax
from jax.experimental import pallas as pl
from jax.experimental.pallas import tpu as pltpu
from jax.experimental.pallas import tpu_sc as plsc

_S = 128
_B = 4096
_D = 256
_WEIGHT = 0.05
_NC = 2
_NS = 16
_NW = _NC * _NS
_RPW = _B // _NW
_L = 16
_G = _RPW // _L


def _sc_body(emb_hbm, sid_hbm, out_sums, out_qc, sid_v, rows_v, acc_s, acc_qc):
    cid = lax.axis_index("c")
    tid = lax.axis_index("s")
    wid = tid * _NC + cid
    base = wid * _RPW

    iota = lax.broadcasted_iota(jnp.int32, (_L,), 0)
    zv = jnp.zeros((_L,), jnp.float32)
    ones = jnp.full((_L,), 1.0, jnp.float32)

    pltpu.sync_copy(sid_hbm.at[pl.ds(base, _RPW)], sid_v)
    pltpu.sync_copy(emb_hbm.at[pl.ds(base, _RPW)], rows_v)

    @plsc.parallel_loop(0, _S, unroll=4)
    def _(r):
        for j in range(_D // _L):
            acc_s[r, pl.ds(j * _L, _L)] = zv
        acc_qc[r, pl.ds(0, _L)] = zv
        acc_qc[r, pl.ds(_L, _L)] = zv

    @plsc.parallel_loop(0, _G, unroll=1)
    def _(g):
        sv = sid_v[pl.ds(g * _L, _L)]
        plsc.addupdate_scatter(acc_qc, [sv, iota + _L], ones)

    @plsc.parallel_loop(0, _G * (_D // _L), unroll=2)
    def _(i):
        g = lax.shift_right_logical(i, 4)
        cb = jnp.bitwise_and(i, _D // _L - 1)
        sv = sid_v[pl.ds(g * _L, _L)]
        rowv = g * _L + iota
        for d in range(_L):
            colv = cb * _L + jnp.bitwise_and(iota + d, _L - 1)
            v = plsc.load_gather(rows_v, [rowv, colv])
            plsc.addupdate_scatter(acc_s, [sv, colv], v)
            plsc.addupdate_scatter(acc_qc, [sv, iota], v * v)

    pltpu.sync_copy(acc_s, out_sums.at[wid])
    pltpu.sync_copy(acc_qc, out_qc.at[wid])


_sc_call = pl.kernel(
    _sc_body,
    out_type=(jax.ShapeDtypeStruct((_NW, _S, _D), jnp.float32),
              jax.ShapeDtypeStruct((_NW, _S, 2 * _L), jnp.float32)),
    mesh=plsc.VectorSubcoreMesh(core_axis_name="c", subcore_axis_name="s"),
    compiler_params=pltpu.CompilerParams(needs_layout_passes=False),
    scratch_types=[
        pltpu.VMEM((_RPW,), jnp.int32),
        pltpu.VMEM((_RPW, _D), jnp.float32),
        pltpu.VMEM((_S, _D), jnp.float32),
        pltpu.VMEM((_S, 2 * _L), jnp.float32),
    ],
)


def _combine_body(sums_ref, qc_ref, out_ref):
    sums = jnp.sum(sums_ref[:], axis=0)
    qc = jnp.sum(qc_ref[:], axis=0)
    ssq = jnp.sum(qc[:, 0:_L], axis=1)
    counts = jnp.sum(qc[:, _L:], axis=1)
    safe = jnp.maximum(counts, 1.0)
    var = (ssq - jnp.sum(sums * sums, axis=1) / safe) / (safe * _D)
    mask = (counts > 1.0).astype(jnp.float32)
    total = jnp.sum(var * mask)
    nsc = jnp.sum(mask)
    loss = jnp.where(nsc > 0, _WEIGHT * total / jnp.maximum(nsc, 1.0), 0.0)
    out_ref[:] = jnp.reshape(loss, (1, 1))


def kernel(embeddings, scaffolds, batch):
    del batch
    sums32, qc = _sc_call(embeddings, scaffolds.astype(jnp.int32))
    out = pl.pallas_call(
        _combine_body,
        out_shape=jax.ShapeDtypeStruct((1, 1), jnp.float32),
    )(sums32, qc)
    return out[0, 0]


# speedup vs baseline: 1.4420x; 1.0763x over previous
import jax
import jax.numpy as jnp
from jax import lax
from jax.experimental import pallas as pl
from jax.experimental.pallas import tpu as pltpu
from jax.experimental.pallas import tpu_sc as plsc

_S = 128
_B = 4096
_D = 256
_WEIGHT = 0.05
_NC = 2
_NS = 16
_NW = _NC * _NS
_RPW = _B // _NW
_L = 16
_G = _RPW // _L


def _sc_body(emb_hbm, sid_hbm, out_sums, out_qc, sid_v, rows_v, acc_s, acc_qc):
    cid = lax.axis_index("c")
    tid = lax.axis_index("s")
    wid = tid * _NC + cid
    base = wid * _RPW

    iota = lax.broadcasted_iota(jnp.int32, (_L,), 0)
    zv = jnp.zeros((_L,), jnp.float32)
    ones = jnp.full((_L,), 1.0, jnp.float32)

    pltpu.sync_copy(sid_hbm.at[pl.ds(base, _RPW)], sid_v)
    pltpu.sync_copy(emb_hbm.at[pl.ds(base, _RPW)], rows_v)

    @plsc.parallel_loop(0, _S, unroll=4)
    def _(r):
        for j in range(_D // _L):
            acc_s[r, pl.ds(j * _L, _L)] = zv
        acc_qc[r, pl.ds(0, _L)] = zv
        acc_qc[r, pl.ds(_L, _L)] = zv

    @plsc.parallel_loop(0, _G, unroll=1)
    def _(g):
        sv = sid_v[pl.ds(g * _L, _L)]
        plsc.addupdate_scatter(acc_qc, [sv, iota + _L], ones)

    @plsc.parallel_loop(0, _RPW, unroll=2)
    def _(b):
        svb = plsc.load_gather(sid_v, [iota * 0 + b])
        sq = []
        for cb in range(_D // _L):
            v = rows_v[b, pl.ds(cb * _L, _L)]
            plsc.addupdate_scatter(acc_s, [svb, cb * _L + iota], v)
            sq.append(v * v)
        while len(sq) > 1:
            sq = [sq[2 * j] + sq[2 * j + 1] for j in range(len(sq) // 2)]
        plsc.addupdate_scatter(acc_qc, [svb, iota], sq[0])

    pltpu.sync_copy(acc_s, out_sums.at[wid])
    pltpu.sync_copy(acc_qc, out_qc.at[wid])


_sc_call = pl.kernel(
    _sc_body,
    out_type=(jax.ShapeDtypeStruct((_NW, _S, _D), jnp.float32),
              jax.ShapeDtypeStruct((_NW, _S, 2 * _L), jnp.float32)),
    mesh=plsc.VectorSubcoreMesh(core_axis_name="c", subcore_axis_name="s"),
    compiler_params=pltpu.CompilerParams(needs_layout_passes=False),
    scratch_types=[
        pltpu.VMEM((_RPW,), jnp.int32),
        pltpu.VMEM((_RPW, _D), jnp.float32),
        pltpu.VMEM((_S, _D), jnp.float32),
        pltpu.VMEM((_S, 2 * _L), jnp.float32),
    ],
)


def _combine_body(sums_ref, qc_ref, out_ref):
    sums = jnp.sum(sums_ref[:], axis=0)
    qc = jnp.sum(qc_ref[:], axis=0)
    ssq = jnp.sum(qc[:, 0:_L], axis=1)
    counts = jnp.sum(qc[:, _L:], axis=1)
    safe = jnp.maximum(counts, 1.0)
    var = (ssq - jnp.sum(sums * sums, axis=1) / safe) / (safe * _D)
    mask = (counts > 1.0).astype(jnp.float32)
    total = jnp.sum(var * mask)
    nsc = jnp.sum(mask)
    loss = jnp.where(nsc > 0, _WEIGHT * total / jnp.maximum(nsc, 1.0), 0.0)
    out_ref[:] = jnp.reshape(loss, (1, 1))


def kernel(embeddings, scaffolds, batch):
    del batch
    sums32, qc = _sc_call(embeddings, scaffolds.astype(jnp.int32))
    out = pl.pallas_call(
        _combine_body,
        out_shape=jax.ShapeDtypeStruct((1, 1), jnp.float32),
    )(sums32, qc)
    return out[0, 0]


# speedup vs baseline: 1.5001x; 1.0403x over previous
import jax
import jax.numpy as jnp
from jax import lax
from jax.experimental import pallas as pl
from jax.experimental.pallas import tpu as pltpu
from jax.experimental.pallas import tpu_sc as plsc

_S = 128
_B = 4096
_D = 256
_WEIGHT = 0.05
_NC = 2
_NS = 16
_NW = _NC * _NS
_RPW = _B // _NW
_L = 16
_G = _RPW // _L


def _sc_body(emb_hbm, sid_hbm, out_sums, out_qc, sid_v, rows_v, acc_s, acc_qc,
             sem):
    cid = lax.axis_index("c")
    tid = lax.axis_index("s")
    wid = tid * _NC + cid
    base = wid * _RPW

    iota = lax.broadcasted_iota(jnp.int32, (_L,), 0)
    zv = jnp.zeros((_L,), jnp.float32)
    ones = jnp.full((_L,), 1.0, jnp.float32)

    pltpu.sync_copy(sid_hbm.at[pl.ds(base, _RPW)], sid_v)
    rows_cp = pltpu.async_copy(emb_hbm.at[pl.ds(base, _RPW)], rows_v, sem)

    @plsc.parallel_loop(0, _S, unroll=4)
    def _(r):
        for j in range(_D // _L):
            acc_s[r, pl.ds(j * _L, _L)] = zv
        acc_qc[r, pl.ds(0, _L)] = zv
        acc_qc[r, pl.ds(_L, _L)] = zv

    @plsc.parallel_loop(0, _G, unroll=1)
    def _(g):
        sv = sid_v[pl.ds(g * _L, _L)]
        plsc.addupdate_scatter(acc_qc, [sv, iota + _L], ones)

    rows_cp.wait()

    @plsc.parallel_loop(0, _RPW, unroll=2)
    def _(b):
        svb = plsc.load_gather(sid_v, [iota * 0 + b])
        sq = []
        for cb in range(_D // _L):
            v = rows_v[b, pl.ds(cb * _L, _L)]
            plsc.addupdate_scatter(acc_s, [svb, cb * _L + iota], v)
            sq.append(v * v)
        while len(sq) > 1:
            sq = [sq[2 * j] + sq[2 * j + 1] for j in range(len(sq) // 2)]
        plsc.addupdate_scatter(acc_qc, [svb, iota], sq[0])

    pltpu.sync_copy(acc_s, out_sums.at[wid])
    pltpu.sync_copy(acc_qc, out_qc.at[wid])


_sc_call = pl.kernel(
    _sc_body,
    out_type=(jax.ShapeDtypeStruct((_NW, _S, _D), jnp.float32),
              jax.ShapeDtypeStruct((_NW, _S, 2 * _L), jnp.float32)),
    mesh=plsc.VectorSubcoreMesh(core_axis_name="c", subcore_axis_name="s"),
    compiler_params=pltpu.CompilerParams(needs_layout_passes=False),
    scratch_types=[
        pltpu.VMEM((_RPW,), jnp.int32),
        pltpu.VMEM((_RPW, _D), jnp.float32),
        pltpu.VMEM((_S, _D), jnp.float32),
        pltpu.VMEM((_S, 2 * _L), jnp.float32),
        pltpu.SemaphoreType.DMA,
    ],
)


def _combine_body(sums_ref, qc_ref, out_ref):
    sums = jnp.sum(sums_ref[:], axis=0)
    qc = jnp.sum(qc_ref[:], axis=0)
    ssq = jnp.sum(qc[:, 0:_L], axis=1)
    counts = jnp.sum(qc[:, _L:], axis=1)
    safe = jnp.maximum(counts, 1.0)
    var = (ssq - jnp.sum(sums * sums, axis=1) / safe) / (safe * _D)
    mask = (counts > 1.0).astype(jnp.float32)
    total = jnp.sum(var * mask)
    nsc = jnp.sum(mask)
    loss = jnp.where(nsc > 0, _WEIGHT * total / jnp.maximum(nsc, 1.0), 0.0)
    out_ref[:] = jnp.reshape(loss, (1, 1))


def kernel(embeddings, scaffolds, batch):
    del batch
    sums32, qc = _sc_call(embeddings, scaffolds.astype(jnp.int32))
    out = pl.pallas_call(
        _combine_body,
        out_shape=jax.ShapeDtypeStruct((1, 1), jnp.float32),
    )(sums32, qc)
    return out[0, 0]
